# baseline (speedup 1.0000x reference)
import jax
import jax.numpy as jnp
from jax.experimental import pallas as pl
from jax.experimental.pallas import tpu as pltpu


def _attn_body(tok_ref, hid_ref, w2_ref, wout_ref, bout_ref, attn_ref, proj_ref):
    b = pl.program_id(0)
    S = hid_ref.shape[1]
    hid = hid_ref[0]
    w2 = w2_ref[...]
    sc = jnp.dot(hid, w2, preferred_element_type=jnp.float32)

    pos = jax.lax.broadcasted_iota(jnp.int32, (S, 1), 0)
    mask = jnp.zeros((S, 1), jnp.bool_)
    for t in range(tok_ref.shape[1]):
        st = tok_ref[b, t, 0]
        en = tok_ref[b, t, 1]
        mask = mask | ((pos >= st) & (pos < en))

    neg = jnp.float32(-1e30)
    scm = jnp.where(mask, sc, neg)
    m = jnp.max(scm, axis=0, keepdims=True)
    m = jnp.where(m > neg * 0.5, m, 0.0)
    e = jnp.where(mask, jnp.exp(sc - m), 0.0)
    denom = jnp.sum(e, axis=0, keepdims=True)
    probs = jnp.where(denom > 0, e / jnp.maximum(denom, 1e-30), 0.0)
    attn_ref[0] = probs.reshape(attn_ref.shape[1], attn_ref.shape[2])

    pooled = jax.lax.dot_general(probs, hid, (((0,), (0,)), ((), ())),
                                 preferred_element_type=jnp.float32)
    proj = jnp.tanh(jnp.dot(pooled.astype(jnp.bfloat16), wout_ref[...],
                            preferred_element_type=jnp.float32) + bout_ref[...])
    proj_ref[pl.ds(b, 1), :] = proj


def kernel(hidden, token_idxs, pooled_entities, W_align, b_align, W_out, b_out):
    B, S, H = hidden.shape
    OUT = W_out.shape[1]
    F = token_idxs.shape[0]
    T = token_idxs.shape[2]
    del pooled_entities, b_align

    tok = token_idxs.reshape(F * B, T, 2).astype(jnp.int32)
    w2 = W_align[H:, :]
    wout_bf = W_out.astype(jnp.bfloat16)
    bout = b_out.reshape(1, OUT)

    attn, proj = pl.pallas_call(
        _attn_body,
        grid=(B,),
        in_specs=[
            pl.BlockSpec(memory_space=pltpu.SMEM),
            pl.BlockSpec((1, S, H), lambda b: (b, 0, 0)),
            pl.BlockSpec((H, 1), lambda b: (0, 0)),
            pl.BlockSpec((H, OUT), lambda b: (0, 0)),
            pl.BlockSpec((1, OUT), lambda b: (0, 0)),
        ],
        out_specs=[
            pl.BlockSpec((1, S // 128, 128), lambda b: (b, 0, 0)),
            pl.BlockSpec((B, OUT), lambda b: (0, 0)),
        ],
        out_shape=[
            jax.ShapeDtypeStruct((B, S // 128, 128), jnp.float32),
            jax.ShapeDtypeStruct((B, OUT), jnp.float32),
        ],
        compiler_params=pltpu.CompilerParams(
            dimension_semantics=("arbitrary",),
            vmem_limit_bytes=100 * 1024 * 1024,
        ),
    )(tok, hidden, w2, wout_bf, bout)

    return proj, attn.reshape(1, B, S, 1)


# speedup vs baseline: 1.1123x; 1.1123x over previous
import jax
import jax.numpy as jnp
from jax.experimental import pallas as pl
from jax.experimental.pallas import tpu as pltpu


def _attn_body(tok_ref, hid_ref, w2_ref, wout_ref, bout_ref, attn_ref, proj_ref):
    b = pl.program_id(0)
    S = hid_ref.shape[1]
    hid = hid_ref[0]
    w2 = w2_ref[...]
    sc = jnp.dot(hid, w2, preferred_element_type=jnp.float32)

    pos = jax.lax.broadcasted_iota(jnp.int32, (S, 1), 0)
    mask = jnp.zeros((S, 1), jnp.bool_)
    for t in range(tok_ref.shape[1]):
        st = tok_ref[b, t, 0]
        en = tok_ref[b, t, 1]
        mask = mask | ((pos >= st) & (pos < en))

    neg = jnp.float32(-1e30)
    scm = jnp.where(mask, sc, neg)
    m = jnp.max(scm, axis=0, keepdims=True)
    m = jnp.where(m > neg * 0.5, m, 0.0)
    e = jnp.where(mask, jnp.exp(sc - m), 0.0)
    denom = jnp.sum(e, axis=0, keepdims=True)
    probs = jnp.where(denom > 0, e / jnp.maximum(denom, 1e-30), 0.0)
    attn_ref[0] = probs.reshape(attn_ref.shape[1], attn_ref.shape[2])

    pooled = jax.lax.dot_general(probs, hid, (((0,), (0,)), ((), ())),
                                 preferred_element_type=jnp.float32)
    proj = jnp.tanh(jnp.dot(pooled, wout_ref[...],
                            preferred_element_type=jnp.float32) + bout_ref[...])
    proj_ref[pl.ds(b, 1), :] = proj


def kernel(hidden, token_idxs, pooled_entities, W_align, b_align, W_out, b_out):
    B, S, H = hidden.shape
    OUT = W_out.shape[1]
    F = token_idxs.shape[0]
    T = token_idxs.shape[2]
    del pooled_entities, b_align

    tok = token_idxs.reshape(F * B, T, 2).astype(jnp.int32)
    w2 = W_align[H:, :]
    bout = b_out.reshape(1, OUT)

    attn, proj = pl.pallas_call(
        _attn_body,
        grid=(B,),
        in_specs=[
            pl.BlockSpec(memory_space=pltpu.SMEM),
            pl.BlockSpec((1, S, H), lambda b: (b, 0, 0)),
            pl.BlockSpec((H, 1), lambda b: (0, 0)),
            pl.BlockSpec((H, OUT), lambda b: (0, 0)),
            pl.BlockSpec((1, OUT), lambda b: (0, 0)),
        ],
        out_specs=[
            pl.BlockSpec((1, S // 128, 128), lambda b: (b, 0, 0)),
            pl.BlockSpec((B, OUT), lambda b: (0, 0)),
        ],
        out_shape=[
            jax.ShapeDtypeStruct((B, S // 128, 128), jnp.float32),
            jax.ShapeDtypeStruct((B, OUT), jnp.float32),
        ],
        compiler_params=pltpu.CompilerParams(
            dimension_semantics=("arbitrary",),
            vmem_limit_bytes=100 * 1024 * 1024,
        ),
    )(tok, hidden, w2, W_out, bout)

    return proj, attn.reshape(1, B, S, 1)
